# baseline (speedup 1.0000x reference)
import jax
import jax.numpy as jnp
from jax.experimental import pallas as pl

HIDDEN = 4096
NUM_EXPERTS = 64
NUM_TOKENS = 32768

BT = 1024


def _router_block(x_ref, wt_ref, b_ref, o_ref):
    o_ref[...] = (
        jnp.dot(x_ref[...], wt_ref[...], preferred_element_type=jnp.float32)
        + b_ref[...]
    )


def kernel(x, W, b):
    wt = W.T
    b2 = b.reshape(1, NUM_EXPERTS)
    grid = (NUM_TOKENS // BT,)
    return pl.pallas_call(
        _router_block,
        grid=grid,
        in_specs=[
            pl.BlockSpec((BT, HIDDEN), lambda i: (i, 0)),
            pl.BlockSpec((HIDDEN, NUM_EXPERTS), lambda i: (0, 0)),
            pl.BlockSpec((1, NUM_EXPERTS), lambda i: (0, 0)),
        ],
        out_specs=pl.BlockSpec((BT, NUM_EXPERTS), lambda i: (i, 0)),
        out_shape=jax.ShapeDtypeStruct((NUM_TOKENS, NUM_EXPERTS), jnp.float32),
    )(x, wt, b2)


# speedup vs baseline: 1.0025x; 1.0025x over previous
import jax
import jax.numpy as jnp
from jax.experimental import pallas as pl
from jax.experimental.pallas import tpu as pltpu

HIDDEN = 4096
NUM_EXPERTS = 64
NUM_TOKENS = 32768

BT = 512


def _router_block(x_ref, wt_ref, b_ref, o_ref):
    o_ref[...] = (
        jnp.dot(x_ref[...], wt_ref[...], preferred_element_type=jnp.float32)
        + b_ref[...]
    )


def kernel(x, W, b):
    wt = W.T
    b2 = b.reshape(1, NUM_EXPERTS)
    grid = (NUM_TOKENS // BT,)
    return pl.pallas_call(
        _router_block,
        grid=grid,
        in_specs=[
            pl.BlockSpec((BT, HIDDEN), lambda i: (i, 0)),
            pl.BlockSpec((HIDDEN, NUM_EXPERTS), lambda i: (0, 0)),
            pl.BlockSpec((1, NUM_EXPERTS), lambda i: (0, 0)),
        ],
        out_specs=pl.BlockSpec((BT, NUM_EXPERTS), lambda i: (i, 0)),
        out_shape=jax.ShapeDtypeStruct((NUM_TOKENS, NUM_EXPERTS), jnp.float32),
        compiler_params=pltpu.CompilerParams(
            dimension_semantics=("parallel",),
        ),
    )(x, wt, b2)
